# baseline (speedup 1.0000x reference)
import jax
import jax.numpy as jnp
from jax import lax
from jax.experimental import pallas as pl
from jax.experimental.pallas import tpu as pltpu
from jax.experimental.pallas import tpu_sc as plsc

F = 100000
V = 50000
NC, NS, L = 2, 16, 16
NT = NC * NS
FT = 3136
FT_LAST = F - (NT - 1) * FT
FT_PAD = 3200
NCHUNK = FT_PAD // 128
VP = 50048
ACC_N = 3 * VP
SLC = ACC_N // NS
PSL = VP // NS
PSL_LAST = V - (NS - 1) * PSL
LAG = 4


def _sc_scatter_body(feat_ref, faces_ref, pa_ref, pb_ref,
                     acc, fk, cb0, cb1, cb2, ix0, ix1, ix2, rbuf, sem):
    c = lax.axis_index("c")
    s = lax.axis_index("s")
    t = s * NC + c

    zeros16f = jnp.zeros((L,), jnp.float32)
    zeros16i = jnp.zeros((L,), jnp.int32)

    def _zf(i, carry):
        cb0[pl.ds(i * L, L)] = zeros16f
        cb1[pl.ds(i * L, L)] = zeros16f
        cb2[pl.ds(i * L, L)] = zeros16f
        fk[pl.ds(i * L, L)] = zeros16i
        return carry
    lax.fori_loop(0, FT_PAD // L, _zf, 0)

    acc_base = pl.multiple_of(s * SLC, 8)
    pltpu.sync_copy(cb0.at[pl.ds(0, 3200)], acc.at[pl.ds(acc_base, 3200)])
    pltpu.sync_copy(cb0.at[pl.ds(0, 3200)], acc.at[pl.ds(acc_base + 3200, 3200)])
    pltpu.sync_copy(cb0.at[pl.ds(0, 2984)], acc.at[pl.ds(acc_base + 6400, 2984)])

    fbase = pl.multiple_of(t * FT, 8)

    @pl.when(t < NT - 1)
    def _():
        pltpu.sync_copy(feat_ref.at[pl.ds(fbase, FT)], cb0.at[pl.ds(0, FT)])
        pltpu.sync_copy(feat_ref.at[pl.ds(F + fbase, FT)], cb1.at[pl.ds(0, FT)])
        pltpu.sync_copy(feat_ref.at[pl.ds(2 * F + fbase, FT)], cb2.at[pl.ds(0, FT)])

    @pl.when(t == NT - 1)
    def _():
        pltpu.sync_copy(feat_ref.at[pl.ds(fbase, FT_LAST)], cb0.at[pl.ds(0, FT_LAST)])
        pltpu.sync_copy(feat_ref.at[pl.ds(F + fbase, FT_LAST)], cb1.at[pl.ds(0, FT_LAST)])
        pltpu.sync_copy(feat_ref.at[pl.ds(2 * F + fbase, FT_LAST)],
                        cb2.at[pl.ds(0, FT_LAST)])

    plsc.subcore_barrier()

    for k in range(3):
        @pl.when(t < NT - 1)
        def _():
            pltpu.sync_copy(faces_ref.at[pl.ds(k * F + fbase, FT)],
                            fk.at[pl.ds(0, FT)])

        @pl.when(t == NT - 1)
        def _():
            pltpu.sync_copy(faces_ref.at[pl.ds(k * F + fbase, FT_LAST)],
                            fk.at[pl.ds(0, FT_LAST)])

        def _build(m, carry):
            for l in range(8):
                v = fk[pl.ds(128 * m + 16 * l, L)]
                ix0[m, pl.ds(l * L, L)] = v
                ix1[m, pl.ds(l * L, L)] = v + VP
                ix2[m, pl.ds(l * L, L)] = v + 2 * VP
            return carry
        lax.fori_loop(0, NCHUNK, _build, 0)

        def _fire(m, carry):
            vbase = pl.multiple_of(m * 128, 8)
            pltpu.async_copy(cb0.at[pl.ds(vbase, 128)], acc.at[ix0.at[m]], sem,
                             add=True)
            pltpu.async_copy(cb1.at[pl.ds(vbase, 128)], acc.at[ix1.at[m]], sem,
                             add=True)
            pltpu.async_copy(cb2.at[pl.ds(vbase, 128)], acc.at[ix2.at[m]], sem,
                             add=True)

            @pl.when(m >= LAG)
            def _():
                pbase = pl.multiple_of((m - LAG) * 128, 8)
                pltpu.make_async_copy(cb0.at[pl.ds(pbase, 128)],
                                      acc.at[ix0.at[m - LAG]], sem).wait()
                pltpu.make_async_copy(cb1.at[pl.ds(pbase, 128)],
                                      acc.at[ix1.at[m - LAG]], sem).wait()
                pltpu.make_async_copy(cb2.at[pl.ds(pbase, 128)],
                                      acc.at[ix2.at[m - LAG]], sem).wait()
            return carry
        lax.fori_loop(0, NCHUNK, _fire, 0)

        def _drain(m, carry):
            pbase = pl.multiple_of(m * 128, 8)
            pltpu.make_async_copy(cb0.at[pl.ds(pbase, 128)],
                                  acc.at[ix0.at[m]], sem).wait()
            pltpu.make_async_copy(cb1.at[pl.ds(pbase, 128)],
                                  acc.at[ix1.at[m]], sem).wait()
            pltpu.make_async_copy(cb2.at[pl.ds(pbase, 128)],
                                  acc.at[ix2.at[m]], sem).wait()
            return carry
        lax.fori_loop(NCHUNK - LAG, NCHUNK, _drain, 0)

    plsc.subcore_barrier()

    p_ref = [pa_ref, pb_ref]
    for ci, pr in enumerate(p_ref):
        @pl.when(c == ci)
        def _():
            for p in range(3):
                src = pl.multiple_of(p * VP + s * PSL, 8)
                dst = pl.multiple_of(p * V + s * PSL, 8)

                @pl.when(s < NS - 1)
                def _():
                    pltpu.sync_copy(acc.at[pl.ds(src, PSL)], rbuf.at[pl.ds(0, PSL)])
                    pltpu.sync_copy(rbuf.at[pl.ds(0, PSL)], pr.at[pl.ds(dst, PSL)])

                @pl.when(s == NS - 1)
                def _():
                    pltpu.sync_copy(acc.at[pl.ds(src, PSL_LAST)],
                                    rbuf.at[pl.ds(0, PSL_LAST)])
                    pltpu.sync_copy(rbuf.at[pl.ds(0, PSL_LAST)],
                                    pr.at[pl.ds(dst, PSL_LAST)])


_sc_scatter = pl.kernel(
    _sc_scatter_body,
    out_type=(jax.ShapeDtypeStruct((3 * V,), jnp.float32),
              jax.ShapeDtypeStruct((3 * V,), jnp.float32)),
    mesh=plsc.VectorSubcoreMesh(core_axis_name="c", subcore_axis_name="s",
                                num_cores=NC, num_subcores=NS),
    compiler_params=pltpu.CompilerParams(needs_layout_passes=False),
    scratch_types=[
        pltpu.VMEM_SHARED((ACC_N,), jnp.float32),
        pltpu.VMEM((FT_PAD,), jnp.int32),
        pltpu.VMEM((FT_PAD,), jnp.float32),
        pltpu.VMEM((FT_PAD,), jnp.float32),
        pltpu.VMEM((FT_PAD,), jnp.float32),
        pltpu.VMEM((NCHUNK, 128), jnp.int32),
        pltpu.VMEM((NCHUNK, 128), jnp.int32),
        pltpu.VMEM((NCHUNK, 128), jnp.int32),
        pltpu.VMEM((PSL,), jnp.float32),
        pltpu.SemaphoreType.DMA,
    ],
)


def _tc_combine_body(rate_ref, vs_ref, pa_ref, pb_ref, o_ref):
    r = rate_ref[0, 0]
    o_ref[...] = vs_ref[...] + r * (pa_ref[...] + pb_ref[...])


_tc_combine = pl.pallas_call(
    _tc_combine_body,
    out_shape=jax.ShapeDtypeStruct((3 * V,), jnp.float32),
    in_specs=[
        pl.BlockSpec(memory_space=pltpu.SMEM),
        pl.BlockSpec(memory_space=pltpu.VMEM),
        pl.BlockSpec(memory_space=pltpu.VMEM),
        pl.BlockSpec(memory_space=pltpu.VMEM),
    ],
)


def kernel(features, vs, faces, rate):
    assert features.shape == (1, 3, F)
    assert vs.shape == (1, V, 3) and faces.shape == (F, 3)

    feat_flat = features.reshape(3 * F)
    faces_sm = jnp.swapaxes(faces, 0, 1).reshape(3 * F)
    pa, pb = _sc_scatter(feat_flat, faces_sm)

    vs_pl = jnp.swapaxes(vs, 1, 2).reshape(3 * V)
    comb = _tc_combine(jnp.asarray(rate, jnp.float32).reshape(1, 1), vs_pl, pa, pb)
    return jnp.swapaxes(comb.reshape(1, 3, V), 1, 2)


# speedup vs baseline: 1.0199x; 1.0199x over previous
import jax
import jax.numpy as jnp
from jax import lax
from jax.experimental import pallas as pl
from jax.experimental.pallas import tpu as pltpu
from jax.experimental.pallas import tpu_sc as plsc

F = 100000
V = 50000
NC, NS, L = 2, 16, 16
NT = NC * NS
FT = 3136
FT_LAST = F - (NT - 1) * FT
FT_PAD = 3200
NCHUNK = FT_PAD // 128
VP = 50048
ACC_N = 3 * VP
SLC = ACC_N // NS
PSL = VP // NS
PSL_LAST = V - (NS - 1) * PSL
LAG = 1


def _sc_scatter_body(feat_ref, faces_ref, pa_ref, pb_ref,
                     acc, fk, cb0, cb1, cb2, ix0, ix1, ix2, rbuf, sem):
    c = lax.axis_index("c")
    s = lax.axis_index("s")
    t = s * NC + c

    zeros16f = jnp.zeros((L,), jnp.float32)
    zeros16i = jnp.zeros((L,), jnp.int32)

    def _zf(i, carry):
        cb0[pl.ds(i * L, L)] = zeros16f
        cb1[pl.ds(i * L, L)] = zeros16f
        cb2[pl.ds(i * L, L)] = zeros16f
        fk[pl.ds(i * L, L)] = zeros16i
        return carry
    lax.fori_loop(0, FT_PAD // L, _zf, 0)

    acc_base = pl.multiple_of(s * SLC, 8)
    pltpu.sync_copy(cb0.at[pl.ds(0, 3200)], acc.at[pl.ds(acc_base, 3200)])
    pltpu.sync_copy(cb0.at[pl.ds(0, 3200)], acc.at[pl.ds(acc_base + 3200, 3200)])
    pltpu.sync_copy(cb0.at[pl.ds(0, 2984)], acc.at[pl.ds(acc_base + 6400, 2984)])

    fbase = pl.multiple_of(t * FT, 8)

    @pl.when(t < NT - 1)
    def _():
        pltpu.sync_copy(feat_ref.at[pl.ds(fbase, FT)], cb0.at[pl.ds(0, FT)])
        pltpu.sync_copy(feat_ref.at[pl.ds(F + fbase, FT)], cb1.at[pl.ds(0, FT)])
        pltpu.sync_copy(feat_ref.at[pl.ds(2 * F + fbase, FT)], cb2.at[pl.ds(0, FT)])

    @pl.when(t == NT - 1)
    def _():
        pltpu.sync_copy(feat_ref.at[pl.ds(fbase, FT_LAST)], cb0.at[pl.ds(0, FT_LAST)])
        pltpu.sync_copy(feat_ref.at[pl.ds(F + fbase, FT_LAST)], cb1.at[pl.ds(0, FT_LAST)])
        pltpu.sync_copy(feat_ref.at[pl.ds(2 * F + fbase, FT_LAST)],
                        cb2.at[pl.ds(0, FT_LAST)])

    plsc.subcore_barrier()

    for k in range(3):
        @pl.when(t < NT - 1)
        def _():
            pltpu.sync_copy(faces_ref.at[pl.ds(k * F + fbase, FT)],
                            fk.at[pl.ds(0, FT)])

        @pl.when(t == NT - 1)
        def _():
            pltpu.sync_copy(faces_ref.at[pl.ds(k * F + fbase, FT_LAST)],
                            fk.at[pl.ds(0, FT_LAST)])

        def _build(m, carry):
            for l in range(8):
                v = fk[pl.ds(128 * m + 16 * l, L)]
                ix0[m, pl.ds(l * L, L)] = v
                ix1[m, pl.ds(l * L, L)] = v + VP
                ix2[m, pl.ds(l * L, L)] = v + 2 * VP
            return carry
        lax.fori_loop(0, NCHUNK, _build, 0)

        def _fire(m, carry):
            vbase = pl.multiple_of(m * 128, 8)
            pltpu.async_copy(cb0.at[pl.ds(vbase, 128)], acc.at[ix0.at[m]], sem,
                             add=True)
            pltpu.async_copy(cb1.at[pl.ds(vbase, 128)], acc.at[ix1.at[m]], sem,
                             add=True)
            pltpu.async_copy(cb2.at[pl.ds(vbase, 128)], acc.at[ix2.at[m]], sem,
                             add=True)

            @pl.when(m >= LAG)
            def _():
                pbase = pl.multiple_of((m - LAG) * 128, 8)
                pltpu.make_async_copy(cb0.at[pl.ds(pbase, 128)],
                                      acc.at[ix0.at[m - LAG]], sem).wait()
                pltpu.make_async_copy(cb1.at[pl.ds(pbase, 128)],
                                      acc.at[ix1.at[m - LAG]], sem).wait()
                pltpu.make_async_copy(cb2.at[pl.ds(pbase, 128)],
                                      acc.at[ix2.at[m - LAG]], sem).wait()
            return carry
        lax.fori_loop(0, NCHUNK, _fire, 0)

        def _drain(m, carry):
            pbase = pl.multiple_of(m * 128, 8)
            pltpu.make_async_copy(cb0.at[pl.ds(pbase, 128)],
                                  acc.at[ix0.at[m]], sem).wait()
            pltpu.make_async_copy(cb1.at[pl.ds(pbase, 128)],
                                  acc.at[ix1.at[m]], sem).wait()
            pltpu.make_async_copy(cb2.at[pl.ds(pbase, 128)],
                                  acc.at[ix2.at[m]], sem).wait()
            return carry
        lax.fori_loop(NCHUNK - LAG, NCHUNK, _drain, 0)

    plsc.subcore_barrier()

    p_ref = [pa_ref, pb_ref]
    for ci, pr in enumerate(p_ref):
        @pl.when(c == ci)
        def _():
            for p in range(3):
                src = pl.multiple_of(p * VP + s * PSL, 8)
                dst = pl.multiple_of(p * V + s * PSL, 8)

                @pl.when(s < NS - 1)
                def _():
                    pltpu.sync_copy(acc.at[pl.ds(src, PSL)], rbuf.at[pl.ds(0, PSL)])
                    pltpu.sync_copy(rbuf.at[pl.ds(0, PSL)], pr.at[pl.ds(dst, PSL)])

                @pl.when(s == NS - 1)
                def _():
                    pltpu.sync_copy(acc.at[pl.ds(src, PSL_LAST)],
                                    rbuf.at[pl.ds(0, PSL_LAST)])
                    pltpu.sync_copy(rbuf.at[pl.ds(0, PSL_LAST)],
                                    pr.at[pl.ds(dst, PSL_LAST)])


_sc_scatter = pl.kernel(
    _sc_scatter_body,
    out_type=(jax.ShapeDtypeStruct((3 * V,), jnp.float32),
              jax.ShapeDtypeStruct((3 * V,), jnp.float32)),
    mesh=plsc.VectorSubcoreMesh(core_axis_name="c", subcore_axis_name="s",
                                num_cores=NC, num_subcores=NS),
    compiler_params=pltpu.CompilerParams(needs_layout_passes=False),
    scratch_types=[
        pltpu.VMEM_SHARED((ACC_N,), jnp.float32),
        pltpu.VMEM((FT_PAD,), jnp.int32),
        pltpu.VMEM((FT_PAD,), jnp.float32),
        pltpu.VMEM((FT_PAD,), jnp.float32),
        pltpu.VMEM((FT_PAD,), jnp.float32),
        pltpu.VMEM((NCHUNK, 128), jnp.int32),
        pltpu.VMEM((NCHUNK, 128), jnp.int32),
        pltpu.VMEM((NCHUNK, 128), jnp.int32),
        pltpu.VMEM((PSL,), jnp.float32),
        pltpu.SemaphoreType.DMA,
    ],
)


def _tc_combine_body(rate_ref, vs_ref, pa_ref, pb_ref, o_ref):
    r = rate_ref[0, 0]
    o_ref[...] = vs_ref[...] + r * (pa_ref[...] + pb_ref[...])


_tc_combine = pl.pallas_call(
    _tc_combine_body,
    out_shape=jax.ShapeDtypeStruct((3 * V,), jnp.float32),
    in_specs=[
        pl.BlockSpec(memory_space=pltpu.SMEM),
        pl.BlockSpec(memory_space=pltpu.VMEM),
        pl.BlockSpec(memory_space=pltpu.VMEM),
        pl.BlockSpec(memory_space=pltpu.VMEM),
    ],
)


def kernel(features, vs, faces, rate):
    assert features.shape == (1, 3, F)
    assert vs.shape == (1, V, 3) and faces.shape == (F, 3)

    feat_flat = features.reshape(3 * F)
    faces_sm = jnp.swapaxes(faces, 0, 1).reshape(3 * F)
    pa, pb = _sc_scatter(feat_flat, faces_sm)

    vs_pl = jnp.swapaxes(vs, 1, 2).reshape(3 * V)
    comb = _tc_combine(jnp.asarray(rate, jnp.float32).reshape(1, 1), vs_pl, pa, pb)
    return jnp.swapaxes(comb.reshape(1, 3, V), 1, 2)


# speedup vs baseline: 1.0263x; 1.0063x over previous
import jax
import jax.numpy as jnp
from jax import lax
from jax.experimental import pallas as pl
from jax.experimental.pallas import tpu as pltpu
from jax.experimental.pallas import tpu_sc as plsc

F = 100000
V = 50000
NC, NS, L = 2, 16, 16
NT = NC * NS
FT = 3136
FT_LAST = F - (NT - 1) * FT
FT_PAD = 3200
NCHUNK = FT_PAD // 128
VP = 50048
ACC_N = 3 * VP
SLC = ACC_N // NS
PSL = VP // NS
PSL_LAST = V - (NS - 1) * PSL
LAG = 1


def _sc_scatter_body(feat_ref, faces_ref, pa_ref, pb_ref,
                     acc, fk, cb0, cb1, cb2, ix0, ix1, ix2, rbuf, sem):
    c = lax.axis_index("c")
    s = lax.axis_index("s")
    t = s * NC + c

    zeros16f = jnp.zeros((L,), jnp.float32)
    zeros16i = jnp.zeros((L,), jnp.int32)

    def _zf(i, carry):
        cb0[pl.ds(i * L, L)] = zeros16f
        cb1[pl.ds(i * L, L)] = zeros16f
        cb2[pl.ds(i * L, L)] = zeros16f
        fk[pl.ds(i * L, L)] = zeros16i
        return carry
    lax.fori_loop(0, FT_PAD // L, _zf, 0)

    acc_base = pl.multiple_of(s * SLC, 8)
    pltpu.sync_copy(cb0.at[pl.ds(0, 3200)], acc.at[pl.ds(acc_base, 3200)])
    pltpu.sync_copy(cb0.at[pl.ds(0, 3200)], acc.at[pl.ds(acc_base + 3200, 3200)])
    pltpu.sync_copy(cb0.at[pl.ds(0, 2984)], acc.at[pl.ds(acc_base + 6400, 2984)])

    fbase = pl.multiple_of(t * FT, 8)

    @pl.when(t < NT - 1)
    def _():
        pltpu.sync_copy(feat_ref.at[pl.ds(fbase, FT)], cb0.at[pl.ds(0, FT)])
        pltpu.sync_copy(feat_ref.at[pl.ds(F + fbase, FT)], cb1.at[pl.ds(0, FT)])
        pltpu.sync_copy(feat_ref.at[pl.ds(2 * F + fbase, FT)], cb2.at[pl.ds(0, FT)])

    @pl.when(t == NT - 1)
    def _():
        pltpu.sync_copy(feat_ref.at[pl.ds(fbase, FT_LAST)], cb0.at[pl.ds(0, FT_LAST)])
        pltpu.sync_copy(feat_ref.at[pl.ds(F + fbase, FT_LAST)], cb1.at[pl.ds(0, FT_LAST)])
        pltpu.sync_copy(feat_ref.at[pl.ds(2 * F + fbase, FT_LAST)],
                        cb2.at[pl.ds(0, FT_LAST)])

    plsc.subcore_barrier()

    def _slot(k, kcarry):
        kbase = pl.multiple_of(k * F + fbase, 8)

        @pl.when(t < NT - 1)
        def _():
            pltpu.sync_copy(faces_ref.at[pl.ds(kbase, FT)],
                            fk.at[pl.ds(0, FT)])

        @pl.when(t == NT - 1)
        def _():
            pltpu.sync_copy(faces_ref.at[pl.ds(kbase, FT_LAST)],
                            fk.at[pl.ds(0, FT_LAST)])

        def _build(m, carry):
            for l in range(8):
                v = fk[pl.ds(128 * m + 16 * l, L)]
                ix0[m, pl.ds(l * L, L)] = v
                ix1[m, pl.ds(l * L, L)] = v + VP
                ix2[m, pl.ds(l * L, L)] = v + 2 * VP
            return carry
        lax.fori_loop(0, NCHUNK, _build, 0)

        def _fire(m, carry):
            vbase = pl.multiple_of(m * 128, 8)
            pltpu.async_copy(cb0.at[pl.ds(vbase, 128)], acc.at[ix0.at[m]], sem,
                             add=True)
            pltpu.async_copy(cb1.at[pl.ds(vbase, 128)], acc.at[ix1.at[m]], sem,
                             add=True)
            pltpu.async_copy(cb2.at[pl.ds(vbase, 128)], acc.at[ix2.at[m]], sem,
                             add=True)

            @pl.when(m >= LAG)
            def _():
                pbase = pl.multiple_of((m - LAG) * 128, 8)
                pltpu.make_async_copy(cb0.at[pl.ds(pbase, 128)],
                                      acc.at[ix0.at[m - LAG]], sem).wait()
                pltpu.make_async_copy(cb1.at[pl.ds(pbase, 128)],
                                      acc.at[ix1.at[m - LAG]], sem).wait()
                pltpu.make_async_copy(cb2.at[pl.ds(pbase, 128)],
                                      acc.at[ix2.at[m - LAG]], sem).wait()
            return carry
        lax.fori_loop(0, NCHUNK, _fire, 0)

        def _drain(m, carry):
            pbase = pl.multiple_of(m * 128, 8)
            pltpu.make_async_copy(cb0.at[pl.ds(pbase, 128)],
                                  acc.at[ix0.at[m]], sem).wait()
            pltpu.make_async_copy(cb1.at[pl.ds(pbase, 128)],
                                  acc.at[ix1.at[m]], sem).wait()
            pltpu.make_async_copy(cb2.at[pl.ds(pbase, 128)],
                                  acc.at[ix2.at[m]], sem).wait()
            return carry
        lax.fori_loop(NCHUNK - LAG, NCHUNK, _drain, 0)
        return kcarry
    lax.fori_loop(0, 3, _slot, 0)

    plsc.subcore_barrier()

    p_ref = [pa_ref, pb_ref]
    for ci, pr in enumerate(p_ref):
        @pl.when(c == ci)
        def _():
            for p in range(3):
                src = pl.multiple_of(p * VP + s * PSL, 8)
                dst = pl.multiple_of(p * V + s * PSL, 8)

                @pl.when(s < NS - 1)
                def _():
                    pltpu.sync_copy(acc.at[pl.ds(src, PSL)], rbuf.at[pl.ds(0, PSL)])
                    pltpu.sync_copy(rbuf.at[pl.ds(0, PSL)], pr.at[pl.ds(dst, PSL)])

                @pl.when(s == NS - 1)
                def _():
                    pltpu.sync_copy(acc.at[pl.ds(src, PSL_LAST)],
                                    rbuf.at[pl.ds(0, PSL_LAST)])
                    pltpu.sync_copy(rbuf.at[pl.ds(0, PSL_LAST)],
                                    pr.at[pl.ds(dst, PSL_LAST)])


_sc_scatter = pl.kernel(
    _sc_scatter_body,
    out_type=(jax.ShapeDtypeStruct((3 * V,), jnp.float32),
              jax.ShapeDtypeStruct((3 * V,), jnp.float32)),
    mesh=plsc.VectorSubcoreMesh(core_axis_name="c", subcore_axis_name="s",
                                num_cores=NC, num_subcores=NS),
    compiler_params=pltpu.CompilerParams(needs_layout_passes=False),
    scratch_types=[
        pltpu.VMEM_SHARED((ACC_N,), jnp.float32),
        pltpu.VMEM((FT_PAD,), jnp.int32),
        pltpu.VMEM((FT_PAD,), jnp.float32),
        pltpu.VMEM((FT_PAD,), jnp.float32),
        pltpu.VMEM((FT_PAD,), jnp.float32),
        pltpu.VMEM((NCHUNK, 128), jnp.int32),
        pltpu.VMEM((NCHUNK, 128), jnp.int32),
        pltpu.VMEM((NCHUNK, 128), jnp.int32),
        pltpu.VMEM((PSL,), jnp.float32),
        pltpu.SemaphoreType.DMA,
    ],
)


def _tc_combine_body(rate_ref, vs_ref, pa_ref, pb_ref, o_ref):
    r = rate_ref[0, 0]
    o_ref[...] = vs_ref[...] + r * (pa_ref[...] + pb_ref[...])


_tc_combine = pl.pallas_call(
    _tc_combine_body,
    out_shape=jax.ShapeDtypeStruct((3 * V,), jnp.float32),
    in_specs=[
        pl.BlockSpec(memory_space=pltpu.SMEM),
        pl.BlockSpec(memory_space=pltpu.VMEM),
        pl.BlockSpec(memory_space=pltpu.VMEM),
        pl.BlockSpec(memory_space=pltpu.VMEM),
    ],
)


def kernel(features, vs, faces, rate):
    assert features.shape == (1, 3, F)
    assert vs.shape == (1, V, 3) and faces.shape == (F, 3)

    feat_flat = features.reshape(3 * F)
    faces_sm = jnp.swapaxes(faces, 0, 1).reshape(3 * F)
    pa, pb = _sc_scatter(feat_flat, faces_sm)

    vs_pl = jnp.swapaxes(vs, 1, 2).reshape(3 * V)
    comb = _tc_combine(jnp.asarray(rate, jnp.float32).reshape(1, 1), vs_pl, pa, pb)
    return jnp.swapaxes(comb.reshape(1, 3, V), 1, 2)
